# baseline (speedup 1.0000x reference)
import functools

import jax
import jax.numpy as jnp
from jax import lax
from jax.experimental import pallas as pl
from jax.experimental.pallas import tpu as pltpu

N_EMBED = 1024
MAX_SEQ_LEN = 8192
BATCH = 4

S_BLK = 256
NUM_BLKS = MAX_SEQ_LEN // S_BLK


def _copy_body(table_ref, out_ref):
    blk = table_ref[...]
    for b in range(BATCH):
        out_ref[b] = blk


@jax.jit
def _tc_broadcast(pos_table):
    return pl.pallas_call(
        _copy_body,
        grid=(NUM_BLKS,),
        in_specs=[
            pl.BlockSpec((S_BLK, N_EMBED), lambda i: (i, 0)),
        ],
        out_specs=pl.BlockSpec((BATCH, S_BLK, N_EMBED), lambda i: (0, i, 0)),
        out_shape=jax.ShapeDtypeStruct((BATCH, MAX_SEQ_LEN, N_EMBED), jnp.float32),
    )(pos_table)


def kernel(input_ids, pos_table):
    del input_ids
    return _tc_broadcast(pos_table)


# speedup vs baseline: 1.0905x; 1.0905x over previous
import functools

import jax
import jax.numpy as jnp
from jax import lax
from jax.experimental import pallas as pl
from jax.experimental.pallas import tpu as pltpu

N_EMBED = 1024
MAX_SEQ_LEN = 8192
BATCH = 4

S_BLK = 1024
NUM_BLKS = MAX_SEQ_LEN // S_BLK


def _copy_body(table_ref, out_ref):
    blk = table_ref[...]
    for b in range(BATCH):
        out_ref[b] = blk


@jax.jit
def _tc_broadcast(pos_table):
    return pl.pallas_call(
        _copy_body,
        grid=(NUM_BLKS,),
        in_specs=[
            pl.BlockSpec((S_BLK, N_EMBED), lambda i: (i, 0)),
        ],
        out_specs=pl.BlockSpec((BATCH, S_BLK, N_EMBED), lambda i: (0, i, 0)),
        out_shape=jax.ShapeDtypeStruct((BATCH, MAX_SEQ_LEN, N_EMBED), jnp.float32),
    )(pos_table)


def kernel(input_ids, pos_table):
    del input_ids
    return _tc_broadcast(pos_table)
